# baseline (speedup 1.0000x reference)
import functools

import jax
import jax.numpy as jnp
from jax import lax
from jax.experimental import pallas as pl
from jax.experimental.pallas import tpu as pltpu
from jax.experimental.pallas import tpu_sc as plsc

HID = 128
NPROT = 5000
G = 4
R = 10368
DUMMY = R - 1
NTILES = 16
CHUNK = R // NTILES
EBATCH = 128
NB = 80
CB = 16
PER_TILE = NB * EBATCH
E_PAD = NTILES * PER_TILE
BR = 256


def _build_indices(mol_edge_index, mol_batch, protein_edge_index, batch_size):
    i32 = jnp.int32
    group = jnp.minimum(mol_batch, batch_size - 1).astype(i32)
    n_mol = group.shape[0]
    grp = jnp.arange(G, dtype=i32)
    cnt = jnp.sum((group[:, None] == grp[None, :]).astype(i32), axis=0)
    end = jnp.cumsum(cnt)
    start = end - cnt
    B1 = end[1] + 2 * NPROT

    def to_flat(p):
        own = (p >= B1).astype(i32)
        return p - own * B1 + own * R

    nodes = jnp.arange(n_mol, dtype=i32)
    mol_pos = nodes + group * NPROT
    ie = group[mol_edge_index[0]]
    src_m = mol_edge_index[0].astype(i32) + ie * NPROT
    dst_m = jnp.clip(mol_edge_index[1].astype(i32) - start[ie], 0,
                     cnt[ie] + NPROT - 1) + start[ie] + ie * NPROT
    center = end[group] + NPROT // 2 + group * NPROT

    S_sm = jnp.concatenate([src_m, mol_pos, center])
    D_sm = jnp.concatenate([dst_m, center, mol_pos])

    pe0 = protein_edge_index[0].astype(i32)
    pe1 = protein_edge_index[1].astype(i32)
    Ep = pe0.shape[0]
    E_SC = S_sm.shape[0] + 2 * Ep
    pad = E_PAD - E_SC

    src_sc, dst_sc = [], []
    for s in (0, 1):
        keep = (D_sm >= B1).astype(i32) == s
        dstloc = jnp.where(keep, D_sm - s * B1, DUMMY)
        srcs = [to_flat(S_sm)]
        dsts = [dstloc]
        for g in (2 * s, 2 * s + 1):
            off = end[g] + g * NPROT
            srcs.append(to_flat(pe0 + off))
            dsts.append(pe1 + off - s * B1)
        src_sc.append(jnp.concatenate(srcs + [jnp.zeros((pad,), i32)]))
        dst_sc.append(jnp.concatenate(dsts + [jnp.full((pad,), DUMMY, i32)]))
    src_all = jnp.concatenate(src_sc).reshape(2 * NTILES * NB, EBATCH)
    dst_all = jnp.concatenate(dst_sc).reshape(2 * NTILES * NB, EBATCH)

    flat_mol = to_flat(mol_pos)
    prot_starts = jnp.stack([to_flat(end[g] + g * NPROT) for g in range(G)])
    gs = jnp.stack([to_flat(start[g] + g * NPROT) for g in range(G)])
    ge = gs + cnt + NPROT
    counts = (cnt + NPROT).astype(jnp.float32)
    return src_all, dst_all, flat_mol, prot_starts, gs, ge, counts


@functools.cache
def _get_sc_scatter():
    mesh = plsc.VectorSubcoreMesh(core_axis_name="c", subcore_axis_name="s")

    @functools.partial(
        pl.kernel,
        mesh=mesh,
        out_type=jax.ShapeDtypeStruct((2 * R, HID), jnp.float32),
        scratch_types=[
            pltpu.VMEM((CB, EBATCH), jnp.int32),
            pltpu.VMEM((CB, EBATCH), jnp.int32),
            pltpu.VMEM((2, EBATCH, HID), jnp.float32),
            pltpu.VMEM_SHARED((R, HID), jnp.float32),
            pltpu.SemaphoreType.DMA,
            pltpu.SemaphoreType.DMA,
        ],
    )
    def _sc_scatter(x_hbm, src_hbm, dst_hbm, zeros_hbm, out_hbm,
                    sidx, didx, rows, acc, sem0, sem1):
        c = lax.axis_index("c")
        t = lax.axis_index("s")
        row0 = (c * NTILES + t) * NB
        pltpu.sync_copy(zeros_hbm, acc.at[pl.ds(t * CHUNK, CHUNK)])
        plsc.subcore_barrier()

        sems = (sem0, sem1)

        def body(ch, carry):
            pltpu.sync_copy(src_hbm.at[pl.ds(row0 + ch * CB, CB)], sidx)
            pltpu.sync_copy(dst_hbm.at[pl.ds(row0 + ch * CB, CB)], didx)
            for b in range(2):
                pltpu.async_copy(x_hbm.at[sidx.at[b]], rows.at[b], sems[b])
            for i in range(CB):
                b = i % 2
                pltpu.make_async_copy(x_hbm.at[sidx.at[i]], rows.at[b],
                                      sems[b]).wait()
                pltpu.sync_copy(rows.at[b], acc.at[didx.at[i]], add=True)
                if i + 2 < CB:
                    pltpu.async_copy(x_hbm.at[sidx.at[i + 2]], rows.at[b],
                                     sems[b])
            return carry

        lax.fori_loop(0, NB // CB, body, 0)
        plsc.subcore_barrier()
        pltpu.sync_copy(acc.at[pl.ds(t * CHUNK, CHUNK)],
                        out_hbm.at[pl.ds(c * R + t * CHUNK, CHUNK)])

    return _sc_scatter


def _mm_bias_body(x_ref, w_ref, b_ref, o_ref):
    o_ref[...] = jnp.dot(x_ref[...], w_ref[...],
                         preferred_element_type=jnp.float32) + b_ref[...]


def _mm_bias(x, w, b, br):
    n = x.shape[0]
    return pl.pallas_call(
        _mm_bias_body,
        grid=(n // br,),
        in_specs=[pl.BlockSpec((br, x.shape[1]), lambda i: (i, 0)),
                  pl.BlockSpec((x.shape[1], HID), lambda i: (0, 0)),
                  pl.BlockSpec((1, HID), lambda i: (0, 0))],
        out_specs=pl.BlockSpec((br, HID), lambda i: (i, 0)),
        out_shape=jax.ShapeDtypeStruct((n, HID), jnp.float32),
    )(x, w, b)


def _mm_scale_body(x_ref, w_ref, deg_ref, o_ref):
    dinv = lax.rsqrt(deg_ref[...] + 1.0)
    o_ref[...] = dinv * jnp.dot(x_ref[...], w_ref[...],
                                preferred_element_type=jnp.float32)


def _mm_scale(x, w, degf):
    return pl.pallas_call(
        _mm_scale_body,
        grid=(2 * R // BR,),
        in_specs=[pl.BlockSpec((BR, HID), lambda i: (i, 0)),
                  pl.BlockSpec((HID, HID), lambda i: (0, 0)),
                  pl.BlockSpec((BR, HID), lambda i: (i, 0))],
        out_specs=pl.BlockSpec((BR, HID), lambda i: (i, 0)),
        out_shape=jax.ShapeDtypeStruct((2 * R, HID), jnp.float32),
    )(x, w, degf)


def _combine_body(s_ref, ht_ref, deg_ref, b_ref, w_ref, o_ref):
    dinv = lax.rsqrt(deg_ref[...] + 1.0)
    x = jnp.maximum(dinv * (s_ref[...] + ht_ref[...]) + b_ref[...], 0.0)
    o_ref[...] = dinv * jnp.dot(x, w_ref[...],
                                preferred_element_type=jnp.float32)


def _combine(s, ht, degf, b, w_next):
    return pl.pallas_call(
        _combine_body,
        grid=(2 * R // BR,),
        in_specs=[pl.BlockSpec((BR, HID), lambda i: (i, 0)),
                  pl.BlockSpec((BR, HID), lambda i: (i, 0)),
                  pl.BlockSpec((BR, HID), lambda i: (i, 0)),
                  pl.BlockSpec((1, HID), lambda i: (0, 0)),
                  pl.BlockSpec((HID, HID), lambda i: (0, 0))],
        out_specs=pl.BlockSpec((BR, HID), lambda i: (i, 0)),
        out_shape=jax.ShapeDtypeStruct((2 * R, HID), jnp.float32),
    )(s, ht, degf, b, w_next)


def _pool_body(m_ref, s_ref, ht_ref, deg_ref, b_ref, o_ref):
    i = pl.program_id(0)
    dinv = lax.rsqrt(deg_ref[...] + 1.0)
    y = dinv * (s_ref[...] + ht_ref[...]) + b_ref[...]

    @pl.when(i == 0)
    def _():
        o_ref[...] = jnp.zeros_like(o_ref)

    o_ref[...] += jnp.dot(m_ref[...], y, preferred_element_type=jnp.float32)


def _pool(s, ht, degf, b, mask):
    return pl.pallas_call(
        _pool_body,
        grid=(2 * R // BR,),
        in_specs=[pl.BlockSpec((8, BR), lambda i: (0, i)),
                  pl.BlockSpec((BR, HID), lambda i: (i, 0)),
                  pl.BlockSpec((BR, HID), lambda i: (i, 0)),
                  pl.BlockSpec((BR, HID), lambda i: (i, 0)),
                  pl.BlockSpec((1, HID), lambda i: (0, 0))],
        out_specs=pl.BlockSpec((8, HID), lambda i: (0, 0)),
        out_shape=jax.ShapeDtypeStruct((8, HID), jnp.float32),
    )(mask, s, ht, degf, b)


def kernel(mol_x, mol_edge_index, mol_batch, protein_x, protein_edge_index,
           batch_size, Wm, bm, Wp, bp, W1, b1, W2, b2, W3, b3):
    f32 = jnp.float32
    src_all, dst_all, flat_mol, prot_starts, gs, ge, counts = _build_indices(
        mol_edge_index, mol_batch, protein_edge_index, batch_size)

    sc_scatter = _get_sc_scatter()
    zeros_chunk = jnp.zeros((CHUNK, HID), f32)
    ones_tab = jnp.ones((2 * R, HID), f32)

    degf = sc_scatter(ones_tab, jnp.zeros_like(src_all), dst_all, zeros_chunk)

    mol_xp = jnp.pad(mol_x.astype(f32), ((0, 0), (0, HID - mol_x.shape[1])))
    Wmp = jnp.pad(Wm.astype(f32), ((0, HID - Wm.shape[0]), (0, 0)))
    prot_xp = jnp.pad(protein_x.astype(f32),
                      ((0, 120), (0, HID - protein_x.shape[1])))
    Wpp = jnp.pad(Wp.astype(f32), ((0, HID - Wp.shape[0]), (0, 0)))
    mol_feats = _mm_bias(mol_xp, Wmp, bm.reshape(1, HID), 320)
    prot_feats = _mm_bias(prot_xp, Wpp, bp.reshape(1, HID), 640)[:NPROT]

    x0 = jnp.zeros((2 * R, HID), f32)
    x0 = x0.at[flat_mol].set(mol_feats)
    for g in range(G):
        x0 = lax.dynamic_update_slice(x0, prot_feats, (prot_starts[g], 0))

    ht = _mm_scale(x0, W1.astype(f32), degf)
    s1 = sc_scatter(ht, src_all, dst_all, zeros_chunk)
    ht = _combine(s1, ht, degf, b1.reshape(1, HID), W2.astype(f32))
    s2 = sc_scatter(ht, src_all, dst_all, zeros_chunk)
    ht = _combine(s2, ht, degf, b2.reshape(1, HID), W3.astype(f32))
    s3 = sc_scatter(ht, src_all, dst_all, zeros_chunk)

    rows = jnp.arange(2 * R, dtype=jnp.int32)
    mask = ((rows[None, :] >= gs[:, None]) &
            (rows[None, :] < ge[:, None])).astype(f32)
    mask = jnp.concatenate([mask, jnp.zeros((8 - G, 2 * R), f32)])
    sums = _pool(s3, ht, degf, b3.reshape(1, HID), mask)[:G]
    return sums / counts[:, None]


# speedup vs baseline: 17.6970x; 17.6970x over previous
import functools

import jax
import jax.numpy as jnp
from jax import lax
from jax.experimental import pallas as pl
from jax.experimental.pallas import tpu as pltpu
from jax.experimental.pallas import tpu_sc as plsc

HID = 128
NPROT = 5000
G = 4
R = 10368
DUMMY = R - 1
NTILES = 16
CHUNK = R // NTILES
EBATCH = 128
NB = 80
CB = 16
PER_TILE = NB * EBATCH
E_PAD = NTILES * PER_TILE
BR = 256


def _build_indices(mol_edge_index, mol_batch, protein_edge_index, batch_size):
    i32 = jnp.int32
    group = jnp.minimum(mol_batch, batch_size - 1).astype(i32)
    n_mol = group.shape[0]
    grp = jnp.arange(G, dtype=i32)
    cnt = jnp.sum((group[:, None] == grp[None, :]).astype(i32), axis=0)
    end = jnp.cumsum(cnt)
    start = end - cnt
    B1 = end[1] + 2 * NPROT

    def to_flat(p):
        own = (p >= B1).astype(i32)
        return p - own * B1 + own * R

    nodes = jnp.arange(n_mol, dtype=i32)
    mol_pos = nodes + group * NPROT
    ie = group[mol_edge_index[0]]
    src_m = mol_edge_index[0].astype(i32) + ie * NPROT
    dst_m = jnp.clip(mol_edge_index[1].astype(i32) - start[ie], 0,
                     cnt[ie] + NPROT - 1) + start[ie] + ie * NPROT
    center = end[group] + NPROT // 2 + group * NPROT

    S_sm = jnp.concatenate([src_m, mol_pos, center])
    D_sm = jnp.concatenate([dst_m, center, mol_pos])

    pe0 = protein_edge_index[0].astype(i32)
    pe1 = protein_edge_index[1].astype(i32)
    Ep = pe0.shape[0]
    E_SC = S_sm.shape[0] + 2 * Ep
    pad = E_PAD - E_SC

    n_sm = S_sm.shape[0]
    spread_sm = 10320 + (jnp.arange(n_sm, dtype=i32) % (R - 10320))
    spread_pad = 10320 + (jnp.arange(pad, dtype=i32) % (R - 10320))
    src_sc, dst_sc = [], []
    for s in (0, 1):
        keep = (D_sm >= B1).astype(i32) == s
        dstloc = jnp.where(keep, D_sm - s * B1, spread_sm)
        srcs = [to_flat(S_sm)]
        dsts = [dstloc]
        for g in (2 * s, 2 * s + 1):
            off = end[g] + g * NPROT
            srcs.append(to_flat(pe0 + off))
            dsts.append(pe1 + off - s * B1)
        pad_src = s * R + 10320 + (jnp.arange(pad, dtype=i32) % (R - 10320))
        tiny_s, tiny_d = srcs[0], dsts[0]
        rest_s = jnp.concatenate(srcs[1:] + [pad_src])
        rest_d = jnp.concatenate(dsts[1:] + [spread_pad])
        n_tiny = tiny_s.shape[0] // NTILES
        n_rest = rest_s.shape[0] // NTILES
        s_cat = jnp.concatenate(
            [tiny_s.reshape(NTILES, n_tiny), rest_s.reshape(NTILES, n_rest)],
            axis=1).reshape(-1)
        d_cat = jnp.concatenate(
            [tiny_d.reshape(NTILES, n_tiny), rest_d.reshape(NTILES, n_rest)],
            axis=1).reshape(-1)
        src_sc.append(s_cat)
        dst_sc.append(d_cat)
    src_all = jnp.concatenate(src_sc).reshape(2 * NTILES * NB, EBATCH)
    dst_all = jnp.concatenate(dst_sc).reshape(2 * NTILES * NB, EBATCH)

    flat_mol = to_flat(mol_pos)
    prot_starts = jnp.stack([to_flat(end[g] + g * NPROT) for g in range(G)])
    gs = jnp.stack([to_flat(start[g] + g * NPROT) for g in range(G)])
    ge = gs + cnt + NPROT
    counts = (cnt + NPROT).astype(jnp.float32)
    return src_all, dst_all, flat_mol, prot_starts, gs, ge, counts


@functools.cache
def _get_sc_scatter():
    mesh = plsc.VectorSubcoreMesh(core_axis_name="c", subcore_axis_name="s")

    @functools.partial(
        pl.kernel,
        mesh=mesh,
        out_type=jax.ShapeDtypeStruct((2 * R, HID), jnp.float32),
        scratch_types=[
            pltpu.VMEM((CB, EBATCH), jnp.int32),
            pltpu.VMEM((CB, EBATCH), jnp.int32),
            pltpu.VMEM((2, EBATCH, HID), jnp.float32),
            pltpu.VMEM_SHARED((R, HID), jnp.float32),
            pltpu.SemaphoreType.DMA,
            pltpu.SemaphoreType.DMA,
        ],
    )
    def _sc_scatter(x_hbm, src_hbm, dst_hbm, zeros_hbm, out_hbm,
                    sidx, didx, rows, acc, sem0, sem1):
        c = lax.axis_index("c")
        t = lax.axis_index("s")
        row0 = (c * NTILES + t) * NB
        pltpu.sync_copy(zeros_hbm, acc.at[pl.ds(t * CHUNK, CHUNK)])
        plsc.subcore_barrier()

        sems = (sem0, sem1)

        def body(ch, carry):
            pltpu.sync_copy(src_hbm.at[pl.ds(row0 + ch * CB, CB)], sidx)
            pltpu.sync_copy(dst_hbm.at[pl.ds(row0 + ch * CB, CB)], didx)
            for b in range(2):
                pltpu.async_copy(x_hbm.at[sidx.at[b]], rows.at[b], sems[b])
            for i in range(CB):
                b = i % 2
                pltpu.make_async_copy(x_hbm.at[sidx.at[i]], rows.at[b],
                                      sems[b]).wait()
                pltpu.sync_copy(rows.at[b], acc.at[didx.at[i]], add=True)
                if i + 2 < CB:
                    pltpu.async_copy(x_hbm.at[sidx.at[i + 2]], rows.at[b],
                                     sems[b])
            return carry

        lax.fori_loop(0, NB // CB, body, 0)
        plsc.subcore_barrier()
        pltpu.sync_copy(acc.at[pl.ds(t * CHUNK, CHUNK)],
                        out_hbm.at[pl.ds(c * R + t * CHUNK, CHUNK)])

    return _sc_scatter


def _mm_bias_body(x_ref, w_ref, b_ref, o_ref):
    o_ref[...] = jnp.dot(x_ref[...], w_ref[...],
                         preferred_element_type=jnp.float32) + b_ref[...]


def _mm_bias(x, w, b, br):
    n = x.shape[0]
    return pl.pallas_call(
        _mm_bias_body,
        grid=(n // br,),
        in_specs=[pl.BlockSpec((br, x.shape[1]), lambda i: (i, 0)),
                  pl.BlockSpec((x.shape[1], HID), lambda i: (0, 0)),
                  pl.BlockSpec((1, HID), lambda i: (0, 0))],
        out_specs=pl.BlockSpec((br, HID), lambda i: (i, 0)),
        out_shape=jax.ShapeDtypeStruct((n, HID), jnp.float32),
    )(x, w, b)


def _mm_scale_body(x_ref, w_ref, deg_ref, o_ref):
    dinv = lax.rsqrt(deg_ref[...] + 1.0)
    o_ref[...] = dinv * jnp.dot(x_ref[...], w_ref[...],
                                preferred_element_type=jnp.float32)


def _mm_scale(x, w, degf):
    return pl.pallas_call(
        _mm_scale_body,
        grid=(2 * R // BR,),
        in_specs=[pl.BlockSpec((BR, HID), lambda i: (i, 0)),
                  pl.BlockSpec((HID, HID), lambda i: (0, 0)),
                  pl.BlockSpec((BR, HID), lambda i: (i, 0))],
        out_specs=pl.BlockSpec((BR, HID), lambda i: (i, 0)),
        out_shape=jax.ShapeDtypeStruct((2 * R, HID), jnp.float32),
    )(x, w, degf)


def _combine_body(s_ref, ht_ref, deg_ref, b_ref, w_ref, o_ref):
    dinv = lax.rsqrt(deg_ref[...] + 1.0)
    x = jnp.maximum(dinv * (s_ref[...] + ht_ref[...]) + b_ref[...], 0.0)
    o_ref[...] = dinv * jnp.dot(x, w_ref[...],
                                preferred_element_type=jnp.float32)


def _combine(s, ht, degf, b, w_next):
    return pl.pallas_call(
        _combine_body,
        grid=(2 * R // BR,),
        in_specs=[pl.BlockSpec((BR, HID), lambda i: (i, 0)),
                  pl.BlockSpec((BR, HID), lambda i: (i, 0)),
                  pl.BlockSpec((BR, HID), lambda i: (i, 0)),
                  pl.BlockSpec((1, HID), lambda i: (0, 0)),
                  pl.BlockSpec((HID, HID), lambda i: (0, 0))],
        out_specs=pl.BlockSpec((BR, HID), lambda i: (i, 0)),
        out_shape=jax.ShapeDtypeStruct((2 * R, HID), jnp.float32),
    )(s, ht, degf, b, w_next)


def _pool_body(m_ref, s_ref, ht_ref, deg_ref, b_ref, o_ref):
    i = pl.program_id(0)
    dinv = lax.rsqrt(deg_ref[...] + 1.0)
    y = dinv * (s_ref[...] + ht_ref[...]) + b_ref[...]

    @pl.when(i == 0)
    def _():
        o_ref[...] = jnp.zeros_like(o_ref)

    o_ref[...] += jnp.dot(m_ref[...], y, preferred_element_type=jnp.float32)


def _pool(s, ht, degf, b, mask):
    return pl.pallas_call(
        _pool_body,
        grid=(2 * R // BR,),
        in_specs=[pl.BlockSpec((8, BR), lambda i: (0, i)),
                  pl.BlockSpec((BR, HID), lambda i: (i, 0)),
                  pl.BlockSpec((BR, HID), lambda i: (i, 0)),
                  pl.BlockSpec((BR, HID), lambda i: (i, 0)),
                  pl.BlockSpec((1, HID), lambda i: (0, 0))],
        out_specs=pl.BlockSpec((8, HID), lambda i: (0, 0)),
        out_shape=jax.ShapeDtypeStruct((8, HID), jnp.float32),
    )(mask, s, ht, degf, b)


def kernel(mol_x, mol_edge_index, mol_batch, protein_x, protein_edge_index,
           batch_size, Wm, bm, Wp, bp, W1, b1, W2, b2, W3, b3):
    f32 = jnp.float32
    src_all, dst_all, flat_mol, prot_starts, gs, ge, counts = _build_indices(
        mol_edge_index, mol_batch, protein_edge_index, batch_size)

    sc_scatter = _get_sc_scatter()
    zeros_chunk = jnp.zeros((CHUNK, HID), f32)
    ones_tab = jnp.ones((2 * R, HID), f32)

    degf = sc_scatter(ones_tab, src_all, dst_all, zeros_chunk)

    mol_xp = jnp.pad(mol_x.astype(f32), ((0, 0), (0, HID - mol_x.shape[1])))
    Wmp = jnp.pad(Wm.astype(f32), ((0, HID - Wm.shape[0]), (0, 0)))
    prot_xp = jnp.pad(protein_x.astype(f32),
                      ((0, 120), (0, HID - protein_x.shape[1])))
    Wpp = jnp.pad(Wp.astype(f32), ((0, HID - Wp.shape[0]), (0, 0)))
    mol_feats = _mm_bias(mol_xp, Wmp, bm.reshape(1, HID), 320)
    prot_feats = _mm_bias(prot_xp, Wpp, bp.reshape(1, HID), 640)[:NPROT]

    x0 = jnp.zeros((2 * R, HID), f32)
    x0 = x0.at[flat_mol].set(mol_feats)
    for g in range(G):
        x0 = lax.dynamic_update_slice(x0, prot_feats, (prot_starts[g], 0))

    ht = _mm_scale(x0, W1.astype(f32), degf)
    s1 = sc_scatter(ht, src_all, dst_all, zeros_chunk)
    ht = _combine(s1, ht, degf, b1.reshape(1, HID), W2.astype(f32))
    s2 = sc_scatter(ht, src_all, dst_all, zeros_chunk)
    ht = _combine(s2, ht, degf, b2.reshape(1, HID), W3.astype(f32))
    s3 = sc_scatter(ht, src_all, dst_all, zeros_chunk)

    rows = jnp.arange(2 * R, dtype=jnp.int32)
    mask = ((rows[None, :] >= gs[:, None]) &
            (rows[None, :] < ge[:, None])).astype(f32)
    mask = jnp.concatenate([mask, jnp.zeros((8 - G, 2 * R), f32)])
    sums = _pool(s3, ht, degf, b3.reshape(1, HID), mask)[:G]
    return sums / counts[:, None]


# speedup vs baseline: 17.8127x; 1.0065x over previous
import functools

import jax
import jax.numpy as jnp
from jax import lax
from jax.experimental import pallas as pl
from jax.experimental.pallas import tpu as pltpu
from jax.experimental.pallas import tpu_sc as plsc

HID = 128
NPROT = 5000
G = 4
R = 10368
DUMMY = R - 1
NTILES = 16
CHUNK = R // NTILES
EBATCH = 128
NB = 80
CB = 16
PER_TILE = NB * EBATCH
E_PAD = NTILES * PER_TILE
BR = 256


def _build_indices(mol_edge_index, mol_batch, protein_edge_index, batch_size):
    i32 = jnp.int32
    group = jnp.minimum(mol_batch, batch_size - 1).astype(i32)
    n_mol = group.shape[0]
    grp = jnp.arange(G, dtype=i32)
    cnt = jnp.sum((group[:, None] == grp[None, :]).astype(i32), axis=0)
    end = jnp.cumsum(cnt)
    start = end - cnt

    nodes = jnp.arange(n_mol, dtype=i32)
    mol_rows = (group // 2) * R + 10000 + nodes
    cent_local = (grp % 2) * NPROT + NPROT // 2
    cent_rows = (grp // 2) * R + cent_local

    ms, md = mol_edge_index[0].astype(i32), mol_edge_index[1].astype(i32)
    ie = group[ms]
    src_mm = (ie // 2) * R + 10000 + ms
    mclip = jnp.maximum(md, start[ie])
    is_mol = md < end[ie]
    dst_mm = jnp.where(is_mol, 10000 + mclip,
                       (ie % 2) * NPROT + (md - end[ie]))
    S_sm = jnp.concatenate([src_mm, mol_rows, cent_rows[group]])
    D_sm = jnp.concatenate([dst_mm, cent_local[group], 10000 + nodes])
    O_sm = jnp.concatenate([ie // 2, group // 2, group // 2])

    pe0 = protein_edge_index[0].astype(i32)
    pe1 = protein_edge_index[1].astype(i32)
    Ep = pe0.shape[0]
    n_sm = S_sm.shape[0]
    pad = E_PAD - (n_sm + 2 * Ep)
    NG = R - 10320
    spread_sm = 10320 + (jnp.arange(n_sm, dtype=i32) % NG)
    spread_pad = 10320 + (jnp.arange(pad, dtype=i32) % NG)

    src_sc, dst_sc = [], []
    for s in (0, 1):
        tiny_d = jnp.where(O_sm == s, D_sm, spread_sm)
        pad_src = s * R + 10320 + (jnp.arange(pad, dtype=i32) % NG)
        rest_s = jnp.concatenate(
            [(s * R) + pe0, (s * R + NPROT) + pe0, pad_src])
        rest_d = jnp.concatenate([pe1, NPROT + pe1, spread_pad])
        n_tiny = n_sm // NTILES
        n_rest = rest_s.shape[0] // NTILES
        src_sc.append(jnp.concatenate(
            [S_sm.reshape(NTILES, n_tiny), rest_s.reshape(NTILES, n_rest)],
            axis=1).reshape(-1))
        dst_sc.append(jnp.concatenate(
            [tiny_d.reshape(NTILES, n_tiny), rest_d.reshape(NTILES, n_rest)],
            axis=1).reshape(-1))
    src_all = jnp.concatenate(src_sc).reshape(2 * NTILES * NB, EBATCH)
    dst_all = jnp.concatenate(dst_sc).reshape(2 * NTILES * NB, EBATCH)

    mask = jnp.zeros((8, 2 * R), jnp.float32)
    for g in range(G):
        s = g // 2
        st = s * R + (g % 2) * NPROT
        mask = mask.at[g, st:st + NPROT].set(1.0)
        molm = (group == g).astype(jnp.float32)
        mask = lax.dynamic_update_slice(mask, molm[None, :],
                                        (g, s * R + 10000))
    counts = (cnt + NPROT).astype(jnp.float32)
    return src_all, dst_all, mask, counts


@functools.cache
def _get_sc_scatter():
    mesh = plsc.VectorSubcoreMesh(core_axis_name="c", subcore_axis_name="s")

    @functools.partial(
        pl.kernel,
        mesh=mesh,
        out_type=jax.ShapeDtypeStruct((2 * R, HID), jnp.float32),
        scratch_types=[
            pltpu.VMEM((CB, EBATCH), jnp.int32),
            pltpu.VMEM((CB, EBATCH), jnp.int32),
            pltpu.VMEM((2, EBATCH, HID), jnp.float32),
            pltpu.VMEM_SHARED((R, HID), jnp.float32),
            pltpu.SemaphoreType.DMA,
            pltpu.SemaphoreType.DMA,
        ],
    )
    def _sc_scatter(x_hbm, src_hbm, dst_hbm, zeros_hbm, out_hbm,
                    sidx, didx, rows, acc, sem0, sem1):
        c = lax.axis_index("c")
        t = lax.axis_index("s")
        row0 = (c * NTILES + t) * NB
        pltpu.sync_copy(zeros_hbm, acc.at[pl.ds(t * CHUNK, CHUNK)])
        plsc.subcore_barrier()

        sems = (sem0, sem1)

        def body(ch, carry):
            pltpu.sync_copy(src_hbm.at[pl.ds(row0 + ch * CB, CB)], sidx)
            pltpu.sync_copy(dst_hbm.at[pl.ds(row0 + ch * CB, CB)], didx)
            for b in range(2):
                pltpu.async_copy(x_hbm.at[sidx.at[b]], rows.at[b], sems[b])
            for i in range(CB):
                b = i % 2
                pltpu.make_async_copy(x_hbm.at[sidx.at[i]], rows.at[b],
                                      sems[b]).wait()
                pltpu.sync_copy(rows.at[b], acc.at[didx.at[i]], add=True)
                if i + 2 < CB:
                    pltpu.async_copy(x_hbm.at[sidx.at[i + 2]], rows.at[b],
                                     sems[b])
            return carry

        lax.fori_loop(0, NB // CB, body, 0)
        plsc.subcore_barrier()
        pltpu.sync_copy(acc.at[pl.ds(t * CHUNK, CHUNK)],
                        out_hbm.at[pl.ds(c * R + t * CHUNK, CHUNK)])

    return _sc_scatter


def _mm_bias_body(x_ref, w_ref, b_ref, o_ref):
    o_ref[...] = jnp.dot(x_ref[...], w_ref[...],
                         preferred_element_type=jnp.float32) + b_ref[...]


def _mm_bias(x, w, b, br):
    n = x.shape[0]
    return pl.pallas_call(
        _mm_bias_body,
        grid=(n // br,),
        in_specs=[pl.BlockSpec((br, x.shape[1]), lambda i: (i, 0)),
                  pl.BlockSpec((x.shape[1], HID), lambda i: (0, 0)),
                  pl.BlockSpec((1, HID), lambda i: (0, 0))],
        out_specs=pl.BlockSpec((br, HID), lambda i: (i, 0)),
        out_shape=jax.ShapeDtypeStruct((n, HID), jnp.float32),
    )(x, w, b)


def _mm_scale_body(x_ref, w_ref, deg_ref, o_ref):
    dinv = lax.rsqrt(deg_ref[...] + 1.0)
    o_ref[...] = dinv * jnp.dot(x_ref[...], w_ref[...],
                                preferred_element_type=jnp.float32)


def _mm_scale(x, w, degf):
    return pl.pallas_call(
        _mm_scale_body,
        grid=(2 * R // BR,),
        in_specs=[pl.BlockSpec((BR, HID), lambda i: (i, 0)),
                  pl.BlockSpec((HID, HID), lambda i: (0, 0)),
                  pl.BlockSpec((BR, HID), lambda i: (i, 0))],
        out_specs=pl.BlockSpec((BR, HID), lambda i: (i, 0)),
        out_shape=jax.ShapeDtypeStruct((2 * R, HID), jnp.float32),
    )(x, w, degf)


def _combine_body(s_ref, ht_ref, deg_ref, b_ref, w_ref, o_ref):
    dinv = lax.rsqrt(deg_ref[...] + 1.0)
    x = jnp.maximum(dinv * (s_ref[...] + ht_ref[...]) + b_ref[...], 0.0)
    o_ref[...] = dinv * jnp.dot(x, w_ref[...],
                                preferred_element_type=jnp.float32)


def _combine(s, ht, degf, b, w_next):
    return pl.pallas_call(
        _combine_body,
        grid=(2 * R // BR,),
        in_specs=[pl.BlockSpec((BR, HID), lambda i: (i, 0)),
                  pl.BlockSpec((BR, HID), lambda i: (i, 0)),
                  pl.BlockSpec((BR, HID), lambda i: (i, 0)),
                  pl.BlockSpec((1, HID), lambda i: (0, 0)),
                  pl.BlockSpec((HID, HID), lambda i: (0, 0))],
        out_specs=pl.BlockSpec((BR, HID), lambda i: (i, 0)),
        out_shape=jax.ShapeDtypeStruct((2 * R, HID), jnp.float32),
    )(s, ht, degf, b, w_next)


def _pool_body(m_ref, s_ref, ht_ref, deg_ref, b_ref, o_ref):
    i = pl.program_id(0)
    dinv = lax.rsqrt(deg_ref[...] + 1.0)
    y = dinv * (s_ref[...] + ht_ref[...]) + b_ref[...]

    @pl.when(i == 0)
    def _():
        o_ref[...] = jnp.zeros_like(o_ref)

    o_ref[...] += jnp.dot(m_ref[...], y, preferred_element_type=jnp.float32)


def _pool(s, ht, degf, b, mask):
    return pl.pallas_call(
        _pool_body,
        grid=(2 * R // BR,),
        in_specs=[pl.BlockSpec((8, BR), lambda i: (0, i)),
                  pl.BlockSpec((BR, HID), lambda i: (i, 0)),
                  pl.BlockSpec((BR, HID), lambda i: (i, 0)),
                  pl.BlockSpec((BR, HID), lambda i: (i, 0)),
                  pl.BlockSpec((1, HID), lambda i: (0, 0))],
        out_specs=pl.BlockSpec((8, HID), lambda i: (0, 0)),
        out_shape=jax.ShapeDtypeStruct((8, HID), jnp.float32),
    )(mask, s, ht, degf, b)


def kernel(mol_x, mol_edge_index, mol_batch, protein_x, protein_edge_index,
           batch_size, Wm, bm, Wp, bp, W1, b1, W2, b2, W3, b3):
    f32 = jnp.float32
    src_all, dst_all, mask, counts = _build_indices(
        mol_edge_index, mol_batch, protein_edge_index, batch_size)

    sc_scatter = _get_sc_scatter()
    zeros_chunk = jnp.zeros((CHUNK, HID), f32)
    ones_tab = jnp.ones((2 * R, HID), f32)

    degf = sc_scatter(ones_tab, src_all, dst_all, zeros_chunk)

    mol_xp = jnp.pad(mol_x.astype(f32), ((0, 0), (0, HID - mol_x.shape[1])))
    Wmp = jnp.pad(Wm.astype(f32), ((0, HID - Wm.shape[0]), (0, 0)))
    prot_xp = jnp.pad(protein_x.astype(f32),
                      ((0, 120), (0, HID - protein_x.shape[1])))
    Wpp = jnp.pad(Wp.astype(f32), ((0, HID - Wp.shape[0]), (0, 0)))
    mol_feats = _mm_bias(mol_xp, Wmp, bm.reshape(1, HID), 320)
    prot_feats = _mm_bias(prot_xp, Wpp, bp.reshape(1, HID), 640)[:NPROT]

    region = jnp.concatenate(
        [prot_feats, prot_feats, mol_feats, jnp.zeros((R - 10320, HID), f32)])
    x0 = jnp.concatenate([region, region])

    ht = _mm_scale(x0, W1.astype(f32), degf)
    s1 = sc_scatter(ht, src_all, dst_all, zeros_chunk)
    ht = _combine(s1, ht, degf, b1.reshape(1, HID), W2.astype(f32))
    s2 = sc_scatter(ht, src_all, dst_all, zeros_chunk)
    ht = _combine(s2, ht, degf, b2.reshape(1, HID), W3.astype(f32))
    s3 = sc_scatter(ht, src_all, dst_all, zeros_chunk)

    sums = _pool(s3, ht, degf, b3.reshape(1, HID), mask)[:G]
    return sums / counts[:, None]


# speedup vs baseline: 20.8105x; 1.1683x over previous
import functools

import jax
import jax.numpy as jnp
from jax import lax
from jax.experimental import pallas as pl
from jax.experimental.pallas import tpu as pltpu
from jax.experimental.pallas import tpu_sc as plsc

HID = 128
NPROT = 5000
G = 4
R = 10368
DUMMY = R - 1
NTILES = 16
CHUNK = R // NTILES
EBATCH = 128
NB = 80
CB = 16
PER_TILE = NB * EBATCH
E_PAD = NTILES * PER_TILE
BR = 768


def _build_indices(mol_edge_index, mol_batch, protein_edge_index, batch_size):
    i32 = jnp.int32
    group = jnp.minimum(mol_batch, batch_size - 1).astype(i32)
    n_mol = group.shape[0]
    grp = jnp.arange(G, dtype=i32)
    cnt = jnp.sum((group[:, None] == grp[None, :]).astype(i32), axis=0)
    end = jnp.cumsum(cnt)
    start = end - cnt

    nodes = jnp.arange(n_mol, dtype=i32)
    mol_rows = (group // 2) * R + 10000 + nodes
    cent_local = (grp % 2) * NPROT + NPROT // 2
    cent_rows = (grp // 2) * R + cent_local

    ms, md = mol_edge_index[0].astype(i32), mol_edge_index[1].astype(i32)
    ie = group[ms]
    src_mm = (ie // 2) * R + 10000 + ms
    mclip = jnp.maximum(md, start[ie])
    is_mol = md < end[ie]
    dst_mm = jnp.where(is_mol, 10000 + mclip,
                       (ie % 2) * NPROT + (md - end[ie]))
    S_sm = jnp.concatenate([src_mm, mol_rows, cent_rows[group]])
    D_sm = jnp.concatenate([dst_mm, cent_local[group], 10000 + nodes])
    O_sm = jnp.concatenate([ie // 2, group // 2, group // 2])

    pe0 = protein_edge_index[0].astype(i32)
    pe1 = protein_edge_index[1].astype(i32)
    Ep = pe0.shape[0]
    n_sm = S_sm.shape[0]
    pad = E_PAD - (n_sm + 2 * Ep)
    NG = R - 10320
    spread_sm = 10320 + (jnp.arange(n_sm, dtype=i32) % NG)
    spread_pad = 10320 + (jnp.arange(pad, dtype=i32) % NG)

    src_sc, dst_sc = [], []
    for s in (0, 1):
        tiny_d = jnp.where(O_sm == s, D_sm, spread_sm)
        pad_src = s * R + 10320 + (jnp.arange(pad, dtype=i32) % NG)
        rest_s = jnp.concatenate(
            [(s * R) + pe0, (s * R + NPROT) + pe0, pad_src])
        rest_d = jnp.concatenate([pe1, NPROT + pe1, spread_pad])
        n_tiny = n_sm // NTILES
        n_rest = rest_s.shape[0] // NTILES
        src_sc.append(jnp.concatenate(
            [S_sm.reshape(NTILES, n_tiny), rest_s.reshape(NTILES, n_rest)],
            axis=1).reshape(-1))
        dst_sc.append(jnp.concatenate(
            [tiny_d.reshape(NTILES, n_tiny), rest_d.reshape(NTILES, n_rest)],
            axis=1).reshape(-1))
    src_all = jnp.concatenate(src_sc).reshape(2 * NTILES * NB, EBATCH)
    dst_all = jnp.concatenate(dst_sc).reshape(2 * NTILES * NB, EBATCH)

    mask = jnp.zeros((8, 2 * R), jnp.float32)
    for g in range(G):
        s = g // 2
        st = s * R + (g % 2) * NPROT
        mask = mask.at[g, st:st + NPROT].set(1.0)
        molm = (group == g).astype(jnp.float32)
        mask = lax.dynamic_update_slice(mask, molm[None, :],
                                        (g, s * R + 10000))
    counts = (cnt + NPROT).astype(jnp.float32)
    return src_all, dst_all, mask, counts


@functools.cache
def _get_sc_scatter():
    mesh = plsc.VectorSubcoreMesh(core_axis_name="c", subcore_axis_name="s")

    @functools.partial(
        pl.kernel,
        mesh=mesh,
        out_type=jax.ShapeDtypeStruct((2 * R, HID), jnp.float32),
        scratch_types=[
            pltpu.VMEM((CB, EBATCH), jnp.int32),
            pltpu.VMEM((CB, EBATCH), jnp.int32),
            pltpu.VMEM((2, EBATCH, HID), jnp.float32),
            pltpu.VMEM_SHARED((R, HID), jnp.float32),
            pltpu.SemaphoreType.DMA,
            pltpu.SemaphoreType.DMA,
        ],
    )
    def _sc_scatter(x_hbm, src_hbm, dst_hbm, zeros_hbm, out_hbm,
                    sidx, didx, rows, acc, sem0, sem1):
        c = lax.axis_index("c")
        t = lax.axis_index("s")
        row0 = (c * NTILES + t) * NB
        pltpu.sync_copy(zeros_hbm, acc.at[pl.ds(t * CHUNK, CHUNK)])
        plsc.subcore_barrier()

        sems = (sem0, sem1)

        def body(ch, carry):
            pltpu.sync_copy(src_hbm.at[pl.ds(row0 + ch * CB, CB)], sidx)
            pltpu.sync_copy(dst_hbm.at[pl.ds(row0 + ch * CB, CB)], didx)
            for b in range(2):
                pltpu.async_copy(x_hbm.at[sidx.at[b]], rows.at[b], sems[b])
            for i in range(CB):
                b = i % 2
                pltpu.make_async_copy(x_hbm.at[sidx.at[i]], rows.at[b],
                                      sems[b]).wait()
                pltpu.sync_copy(rows.at[b], acc.at[didx.at[i]], add=True)
                if i + 2 < CB:
                    pltpu.async_copy(x_hbm.at[sidx.at[i + 2]], rows.at[b],
                                     sems[b])
            return carry

        lax.fori_loop(0, NB // CB, body, 0)
        plsc.subcore_barrier()
        pltpu.sync_copy(acc.at[pl.ds(t * CHUNK, CHUNK)],
                        out_hbm.at[pl.ds(c * R + t * CHUNK, CHUNK)])

    return _sc_scatter


def _mm_bias_body(x_ref, w_ref, b_ref, o_ref):
    o_ref[...] = jnp.dot(x_ref[...], w_ref[...],
                         preferred_element_type=jnp.float32) + b_ref[...]


def _mm_bias(x, w, b, br):
    n = x.shape[0]
    return pl.pallas_call(
        _mm_bias_body,
        grid=(n // br,),
        in_specs=[pl.BlockSpec((br, x.shape[1]), lambda i: (i, 0)),
                  pl.BlockSpec((x.shape[1], HID), lambda i: (0, 0)),
                  pl.BlockSpec((1, HID), lambda i: (0, 0))],
        out_specs=pl.BlockSpec((br, HID), lambda i: (i, 0)),
        out_shape=jax.ShapeDtypeStruct((n, HID), jnp.float32),
    )(x, w, b)


def _mm_scale_body(x_ref, w_ref, deg_ref, o_ref):
    dinv = lax.rsqrt(deg_ref[...] + 1.0)
    o_ref[...] = dinv * jnp.dot(x_ref[...], w_ref[...],
                                preferred_element_type=jnp.float32)


def _mm_scale(x, w, degf):
    return pl.pallas_call(
        _mm_scale_body,
        grid=(2 * R // BR,),
        in_specs=[pl.BlockSpec((BR, HID), lambda i: (i, 0)),
                  pl.BlockSpec((HID, HID), lambda i: (0, 0)),
                  pl.BlockSpec((BR, HID), lambda i: (i, 0))],
        out_specs=pl.BlockSpec((BR, HID), lambda i: (i, 0)),
        out_shape=jax.ShapeDtypeStruct((2 * R, HID), jnp.float32),
    )(x, w, degf)


def _combine_body(s_ref, ht_ref, deg_ref, b_ref, w_ref, o_ref):
    dinv = lax.rsqrt(deg_ref[...] + 1.0)
    x = jnp.maximum(dinv * (s_ref[...] + ht_ref[...]) + b_ref[...], 0.0)
    o_ref[...] = dinv * jnp.dot(x, w_ref[...],
                                preferred_element_type=jnp.float32)


def _combine(s, ht, degf, b, w_next):
    return pl.pallas_call(
        _combine_body,
        grid=(2 * R // BR,),
        in_specs=[pl.BlockSpec((BR, HID), lambda i: (i, 0)),
                  pl.BlockSpec((BR, HID), lambda i: (i, 0)),
                  pl.BlockSpec((BR, HID), lambda i: (i, 0)),
                  pl.BlockSpec((1, HID), lambda i: (0, 0)),
                  pl.BlockSpec((HID, HID), lambda i: (0, 0))],
        out_specs=pl.BlockSpec((BR, HID), lambda i: (i, 0)),
        out_shape=jax.ShapeDtypeStruct((2 * R, HID), jnp.float32),
    )(s, ht, degf, b, w_next)


def _pool_body(m_ref, s_ref, ht_ref, deg_ref, b_ref, o_ref):
    i = pl.program_id(0)
    dinv = lax.rsqrt(deg_ref[...] + 1.0)
    y = dinv * (s_ref[...] + ht_ref[...]) + b_ref[...]

    @pl.when(i == 0)
    def _():
        o_ref[...] = jnp.zeros_like(o_ref)

    o_ref[...] += jnp.dot(m_ref[...], y, preferred_element_type=jnp.float32)


def _pool(s, ht, degf, b, mask):
    return pl.pallas_call(
        _pool_body,
        grid=(2 * R // BR,),
        in_specs=[pl.BlockSpec((8, BR), lambda i: (0, i)),
                  pl.BlockSpec((BR, HID), lambda i: (i, 0)),
                  pl.BlockSpec((BR, HID), lambda i: (i, 0)),
                  pl.BlockSpec((BR, HID), lambda i: (i, 0)),
                  pl.BlockSpec((1, HID), lambda i: (0, 0))],
        out_specs=pl.BlockSpec((8, HID), lambda i: (0, 0)),
        out_shape=jax.ShapeDtypeStruct((8, HID), jnp.float32),
    )(mask, s, ht, degf, b)


def kernel(mol_x, mol_edge_index, mol_batch, protein_x, protein_edge_index,
           batch_size, Wm, bm, Wp, bp, W1, b1, W2, b2, W3, b3):
    f32 = jnp.float32
    src_all, dst_all, mask, counts = _build_indices(
        mol_edge_index, mol_batch, protein_edge_index, batch_size)

    sc_scatter = _get_sc_scatter()
    zeros_chunk = jnp.zeros((CHUNK, HID), f32)
    ones_tab = jnp.ones((2 * R, HID), f32)

    degf = sc_scatter(ones_tab, src_all, dst_all, zeros_chunk)

    mol_xp = jnp.pad(mol_x.astype(f32), ((0, 0), (0, HID - mol_x.shape[1])))
    Wmp = jnp.pad(Wm.astype(f32), ((0, HID - Wm.shape[0]), (0, 0)))
    prot_xp = jnp.pad(protein_x.astype(f32),
                      ((0, 120), (0, HID - protein_x.shape[1])))
    Wpp = jnp.pad(Wp.astype(f32), ((0, HID - Wp.shape[0]), (0, 0)))
    mol_feats = _mm_bias(mol_xp, Wmp, bm.reshape(1, HID), 320)
    prot_feats = _mm_bias(prot_xp, Wpp, bp.reshape(1, HID), 640)[:NPROT]

    region = jnp.concatenate(
        [prot_feats, prot_feats, mol_feats, jnp.zeros((R - 10320, HID), f32)])
    x0 = jnp.concatenate([region, region])

    ht = _mm_scale(x0, W1.astype(f32), degf)
    s1 = sc_scatter(ht, src_all, dst_all, zeros_chunk)
    ht = _combine(s1, ht, degf, b1.reshape(1, HID), W2.astype(f32))
    s2 = sc_scatter(ht, src_all, dst_all, zeros_chunk)
    ht = _combine(s2, ht, degf, b2.reshape(1, HID), W3.astype(f32))
    s3 = sc_scatter(ht, src_all, dst_all, zeros_chunk)

    sums = _pool(s3, ht, degf, b3.reshape(1, HID), mask)[:G]
    return sums / counts[:, None]


# speedup vs baseline: 22.0414x; 1.0591x over previous
import functools

import jax
import jax.numpy as jnp
from jax import lax
from jax.experimental import pallas as pl
from jax.experimental.pallas import tpu as pltpu
from jax.experimental.pallas import tpu_sc as plsc

HID = 128
NPROT = 5000
G = 4
R = 10368
DUMMY = R - 1
NTILES = 16
CHUNK = R // NTILES
EBATCH = 128
NB = 80
CB = 16
PER_TILE = NB * EBATCH
E_PAD = NTILES * PER_TILE
BR = 2304


def _build_indices(mol_edge_index, mol_batch, protein_edge_index, batch_size):
    i32 = jnp.int32
    group = jnp.minimum(mol_batch, batch_size - 1).astype(i32)
    n_mol = group.shape[0]
    grp = jnp.arange(G, dtype=i32)
    cnt = jnp.sum((group[:, None] == grp[None, :]).astype(i32), axis=0)
    end = jnp.cumsum(cnt)
    start = end - cnt

    nodes = jnp.arange(n_mol, dtype=i32)
    mol_rows = (group // 2) * R + 10000 + nodes
    cent_local = (grp % 2) * NPROT + NPROT // 2
    cent_rows = (grp // 2) * R + cent_local

    ms, md = mol_edge_index[0].astype(i32), mol_edge_index[1].astype(i32)
    ie = group[ms]
    src_mm = (ie // 2) * R + 10000 + ms
    mclip = jnp.maximum(md, start[ie])
    is_mol = md < end[ie]
    dst_mm = jnp.where(is_mol, 10000 + mclip,
                       (ie % 2) * NPROT + (md - end[ie]))
    S_sm = jnp.concatenate([src_mm, mol_rows, cent_rows[group]])
    D_sm = jnp.concatenate([dst_mm, cent_local[group], 10000 + nodes])
    O_sm = jnp.concatenate([ie // 2, group // 2, group // 2])

    pe0 = protein_edge_index[0].astype(i32)
    pe1 = protein_edge_index[1].astype(i32)
    Ep = pe0.shape[0]
    n_sm = S_sm.shape[0]
    pad = E_PAD - (n_sm + 2 * Ep)
    NG = R - 10320
    spread_sm = 10320 + (jnp.arange(n_sm, dtype=i32) % NG)
    spread_pad = 10320 + (jnp.arange(pad, dtype=i32) % NG)

    src_sc, dst_sc = [], []
    for s in (0, 1):
        tiny_d = jnp.where(O_sm == s, D_sm, spread_sm)
        pad_src = s * R + 10320 + (jnp.arange(pad, dtype=i32) % NG)
        rest_s = jnp.concatenate(
            [(s * R) + pe0, (s * R + NPROT) + pe0, pad_src])
        rest_d = jnp.concatenate([pe1, NPROT + pe1, spread_pad])
        n_tiny = n_sm // NTILES
        n_rest = rest_s.shape[0] // NTILES
        src_sc.append(jnp.concatenate(
            [S_sm.reshape(NTILES, n_tiny), rest_s.reshape(NTILES, n_rest)],
            axis=1).reshape(-1))
        dst_sc.append(jnp.concatenate(
            [tiny_d.reshape(NTILES, n_tiny), rest_d.reshape(NTILES, n_rest)],
            axis=1).reshape(-1))
    src_all = jnp.concatenate(src_sc).reshape(2 * NTILES * NB, EBATCH)
    dst_all = jnp.concatenate(dst_sc).reshape(2 * NTILES * NB, EBATCH)

    mask = jnp.zeros((8, 2 * R), jnp.float32)
    for g in range(G):
        s = g // 2
        st = s * R + (g % 2) * NPROT
        mask = mask.at[g, st:st + NPROT].set(1.0)
        molm = (group == g).astype(jnp.float32)
        mask = lax.dynamic_update_slice(mask, molm[None, :],
                                        (g, s * R + 10000))
    counts = (cnt + NPROT).astype(jnp.float32)
    return src_all, dst_all, mask, counts


@functools.cache
def _get_sc_scatter():
    mesh = plsc.VectorSubcoreMesh(core_axis_name="c", subcore_axis_name="s")

    @functools.partial(
        pl.kernel,
        mesh=mesh,
        out_type=jax.ShapeDtypeStruct((2 * R, HID), jnp.float32),
        scratch_types=[
            pltpu.VMEM((CB, EBATCH), jnp.int32),
            pltpu.VMEM((CB, EBATCH), jnp.int32),
            pltpu.VMEM((2, EBATCH, HID), jnp.float32),
            pltpu.VMEM_SHARED((R, HID), jnp.float32),
            pltpu.SemaphoreType.DMA,
            pltpu.SemaphoreType.DMA,
        ],
    )
    def _sc_scatter(x_hbm, src_hbm, dst_hbm, zeros_hbm, out_hbm,
                    sidx, didx, rows, acc, sem0, sem1):
        c = lax.axis_index("c")
        t = lax.axis_index("s")
        row0 = (c * NTILES + t) * NB
        pltpu.sync_copy(zeros_hbm, acc.at[pl.ds(t * CHUNK, CHUNK)])
        plsc.subcore_barrier()

        sems = (sem0, sem1)

        def body(ch, carry):
            pltpu.sync_copy(src_hbm.at[pl.ds(row0 + ch * CB, CB)], sidx)
            pltpu.sync_copy(dst_hbm.at[pl.ds(row0 + ch * CB, CB)], didx)
            for b in range(2):
                pltpu.async_copy(x_hbm.at[sidx.at[b]], rows.at[b], sems[b])
            for i in range(CB):
                b = i % 2
                pltpu.make_async_copy(x_hbm.at[sidx.at[i]], rows.at[b],
                                      sems[b]).wait()
                pltpu.sync_copy(rows.at[b], acc.at[didx.at[i]], add=True)
                if i + 2 < CB:
                    pltpu.async_copy(x_hbm.at[sidx.at[i + 2]], rows.at[b],
                                     sems[b])
            return carry

        lax.fori_loop(0, NB // CB, body, 0)
        plsc.subcore_barrier()
        pltpu.sync_copy(acc.at[pl.ds(t * CHUNK, CHUNK)],
                        out_hbm.at[pl.ds(c * R + t * CHUNK, CHUNK)])

    return _sc_scatter


def _mm_bias_body(x_ref, w_ref, b_ref, o_ref):
    o_ref[...] = jnp.dot(x_ref[...], w_ref[...],
                         preferred_element_type=jnp.float32) + b_ref[...]


def _mm_bias(x, w, b, br):
    n = x.shape[0]
    return pl.pallas_call(
        _mm_bias_body,
        grid=(n // br,),
        in_specs=[pl.BlockSpec((br, x.shape[1]), lambda i: (i, 0)),
                  pl.BlockSpec((x.shape[1], HID), lambda i: (0, 0)),
                  pl.BlockSpec((1, HID), lambda i: (0, 0))],
        out_specs=pl.BlockSpec((br, HID), lambda i: (i, 0)),
        out_shape=jax.ShapeDtypeStruct((n, HID), jnp.float32),
    )(x, w, b)


def _mm_scale_body(x_ref, w_ref, deg_ref, o_ref):
    dinv = lax.rsqrt(deg_ref[...] + 1.0)
    o_ref[...] = dinv * jnp.dot(x_ref[...], w_ref[...],
                                preferred_element_type=jnp.float32)


def _mm_scale(x, w, degf):
    return pl.pallas_call(
        _mm_scale_body,
        grid=(2 * R // BR,),
        in_specs=[pl.BlockSpec((BR, HID), lambda i: (i, 0)),
                  pl.BlockSpec((HID, HID), lambda i: (0, 0)),
                  pl.BlockSpec((BR, HID), lambda i: (i, 0))],
        out_specs=pl.BlockSpec((BR, HID), lambda i: (i, 0)),
        out_shape=jax.ShapeDtypeStruct((2 * R, HID), jnp.float32),
    )(x, w, degf)


def _combine_body(s_ref, ht_ref, deg_ref, b_ref, w_ref, o_ref):
    dinv = lax.rsqrt(deg_ref[...] + 1.0)
    x = jnp.maximum(dinv * (s_ref[...] + ht_ref[...]) + b_ref[...], 0.0)
    o_ref[...] = dinv * jnp.dot(x, w_ref[...],
                                preferred_element_type=jnp.float32)


def _combine(s, ht, degf, b, w_next):
    return pl.pallas_call(
        _combine_body,
        grid=(2 * R // BR,),
        in_specs=[pl.BlockSpec((BR, HID), lambda i: (i, 0)),
                  pl.BlockSpec((BR, HID), lambda i: (i, 0)),
                  pl.BlockSpec((BR, HID), lambda i: (i, 0)),
                  pl.BlockSpec((1, HID), lambda i: (0, 0)),
                  pl.BlockSpec((HID, HID), lambda i: (0, 0))],
        out_specs=pl.BlockSpec((BR, HID), lambda i: (i, 0)),
        out_shape=jax.ShapeDtypeStruct((2 * R, HID), jnp.float32),
    )(s, ht, degf, b, w_next)


def _pool_body(m_ref, s_ref, ht_ref, deg_ref, b_ref, o_ref):
    i = pl.program_id(0)
    dinv = lax.rsqrt(deg_ref[...] + 1.0)
    y = dinv * (s_ref[...] + ht_ref[...]) + b_ref[...]

    @pl.when(i == 0)
    def _():
        o_ref[...] = jnp.zeros_like(o_ref)

    o_ref[...] += jnp.dot(m_ref[...], y, preferred_element_type=jnp.float32)


def _pool(s, ht, degf, b, mask):
    return pl.pallas_call(
        _pool_body,
        grid=(2 * R // BR,),
        in_specs=[pl.BlockSpec((8, BR), lambda i: (0, i)),
                  pl.BlockSpec((BR, HID), lambda i: (i, 0)),
                  pl.BlockSpec((BR, HID), lambda i: (i, 0)),
                  pl.BlockSpec((BR, HID), lambda i: (i, 0)),
                  pl.BlockSpec((1, HID), lambda i: (0, 0))],
        out_specs=pl.BlockSpec((8, HID), lambda i: (0, 0)),
        out_shape=jax.ShapeDtypeStruct((8, HID), jnp.float32),
    )(mask, s, ht, degf, b)


def kernel(mol_x, mol_edge_index, mol_batch, protein_x, protein_edge_index,
           batch_size, Wm, bm, Wp, bp, W1, b1, W2, b2, W3, b3):
    f32 = jnp.float32
    src_all, dst_all, mask, counts = _build_indices(
        mol_edge_index, mol_batch, protein_edge_index, batch_size)

    sc_scatter = _get_sc_scatter()
    zeros_chunk = jnp.zeros((CHUNK, HID), f32)
    ones_tab = jnp.ones((2 * R, HID), f32)

    degf = sc_scatter(ones_tab, src_all, dst_all, zeros_chunk)

    mol_xp = jnp.pad(mol_x.astype(f32), ((0, 0), (0, HID - mol_x.shape[1])))
    Wmp = jnp.pad(Wm.astype(f32), ((0, HID - Wm.shape[0]), (0, 0)))
    prot_xp = jnp.pad(protein_x.astype(f32),
                      ((0, 120), (0, HID - protein_x.shape[1])))
    Wpp = jnp.pad(Wp.astype(f32), ((0, HID - Wp.shape[0]), (0, 0)))
    mol_feats = _mm_bias(mol_xp, Wmp, bm.reshape(1, HID), 320)
    prot_feats = _mm_bias(prot_xp, Wpp, bp.reshape(1, HID), 640)[:NPROT]

    region = jnp.concatenate(
        [prot_feats, prot_feats, mol_feats, jnp.zeros((R - 10320, HID), f32)])
    x0 = jnp.concatenate([region, region])

    ht = _mm_scale(x0, W1.astype(f32), degf)
    s1 = sc_scatter(ht, src_all, dst_all, zeros_chunk)
    ht = _combine(s1, ht, degf, b1.reshape(1, HID), W2.astype(f32))
    s2 = sc_scatter(ht, src_all, dst_all, zeros_chunk)
    ht = _combine(s2, ht, degf, b2.reshape(1, HID), W3.astype(f32))
    s3 = sc_scatter(ht, src_all, dst_all, zeros_chunk)

    sums = _pool(s3, ht, degf, b3.reshape(1, HID), mask)[:G]
    return sums / counts[:, None]


# speedup vs baseline: 22.2082x; 1.0076x over previous
import functools

import jax
import jax.numpy as jnp
from jax import lax
from jax.experimental import pallas as pl
from jax.experimental.pallas import tpu as pltpu
from jax.experimental.pallas import tpu_sc as plsc

HID = 128
NPROT = 5000
G = 4
R = 10368
DUMMY = R - 1
NTILES = 16
CHUNK = R // NTILES
EBATCH = 128
NB = 80
CB = 16
PER_TILE = NB * EBATCH
E_PAD = NTILES * PER_TILE
BR = 3456


def _build_indices(mol_edge_index, mol_batch, protein_edge_index, batch_size):
    i32 = jnp.int32
    group = jnp.minimum(mol_batch, batch_size - 1).astype(i32)
    n_mol = group.shape[0]
    grp = jnp.arange(G, dtype=i32)
    cnt = jnp.sum((group[:, None] == grp[None, :]).astype(i32), axis=0)
    end = jnp.cumsum(cnt)
    start = end - cnt

    nodes = jnp.arange(n_mol, dtype=i32)
    mol_rows = (group // 2) * R + 10000 + nodes
    cent_local = (grp % 2) * NPROT + NPROT // 2
    cent_rows = (grp // 2) * R + cent_local

    ms, md = mol_edge_index[0].astype(i32), mol_edge_index[1].astype(i32)
    ie = group[ms]
    src_mm = (ie // 2) * R + 10000 + ms
    mclip = jnp.maximum(md, start[ie])
    is_mol = md < end[ie]
    dst_mm = jnp.where(is_mol, 10000 + mclip,
                       (ie % 2) * NPROT + (md - end[ie]))
    S_sm = jnp.concatenate([src_mm, mol_rows, cent_rows[group]])
    D_sm = jnp.concatenate([dst_mm, cent_local[group], 10000 + nodes])
    O_sm = jnp.concatenate([ie // 2, group // 2, group // 2])

    pe0 = protein_edge_index[0].astype(i32)
    pe1 = protein_edge_index[1].astype(i32)
    Ep = pe0.shape[0]
    n_sm = S_sm.shape[0]
    pad = E_PAD - (n_sm + 2 * Ep)
    NG = R - 10320
    spread_sm = 10320 + (jnp.arange(n_sm, dtype=i32) % NG)
    spread_pad = 10320 + (jnp.arange(pad, dtype=i32) % NG)

    src_sc, dst_sc = [], []
    for s in (0, 1):
        tiny_d = jnp.where(O_sm == s, D_sm, spread_sm)
        pad_src = s * R + 10320 + (jnp.arange(pad, dtype=i32) % NG)
        rest_s = jnp.concatenate(
            [(s * R) + pe0, (s * R + NPROT) + pe0, pad_src])
        rest_d = jnp.concatenate([pe1, NPROT + pe1, spread_pad])
        n_tiny = n_sm // NTILES
        n_rest = rest_s.shape[0] // NTILES
        src_sc.append(jnp.concatenate(
            [S_sm.reshape(NTILES, n_tiny), rest_s.reshape(NTILES, n_rest)],
            axis=1).reshape(-1))
        dst_sc.append(jnp.concatenate(
            [tiny_d.reshape(NTILES, n_tiny), rest_d.reshape(NTILES, n_rest)],
            axis=1).reshape(-1))
    src_all = jnp.concatenate(src_sc).reshape(2 * NTILES * NB, EBATCH)
    dst_all = jnp.concatenate(dst_sc).reshape(2 * NTILES * NB, EBATCH)

    mask = jnp.zeros((8, 2 * R), jnp.float32)
    for g in range(G):
        s = g // 2
        st = s * R + (g % 2) * NPROT
        mask = mask.at[g, st:st + NPROT].set(1.0)
        molm = (group == g).astype(jnp.float32)
        mask = lax.dynamic_update_slice(mask, molm[None, :],
                                        (g, s * R + 10000))
    counts = (cnt + NPROT).astype(jnp.float32)
    return src_all, dst_all, mask, counts


@functools.cache
def _get_sc_scatter():
    mesh = plsc.VectorSubcoreMesh(core_axis_name="c", subcore_axis_name="s")

    @functools.partial(
        pl.kernel,
        mesh=mesh,
        out_type=jax.ShapeDtypeStruct((2 * R, HID), jnp.float32),
        scratch_types=[
            pltpu.VMEM((CB, EBATCH), jnp.int32),
            pltpu.VMEM((CB, EBATCH), jnp.int32),
            pltpu.VMEM((2, EBATCH, HID), jnp.float32),
            pltpu.VMEM_SHARED((R, HID), jnp.float32),
            pltpu.SemaphoreType.DMA,
            pltpu.SemaphoreType.DMA,
        ],
    )
    def _sc_scatter(x_hbm, src_hbm, dst_hbm, zeros_hbm, out_hbm,
                    sidx, didx, rows, acc, sem0, sem1):
        c = lax.axis_index("c")
        t = lax.axis_index("s")
        row0 = (c * NTILES + t) * NB
        pltpu.sync_copy(zeros_hbm, acc.at[pl.ds(t * CHUNK, CHUNK)])
        plsc.subcore_barrier()

        sems = (sem0, sem1)

        def body(ch, carry):
            pltpu.sync_copy(src_hbm.at[pl.ds(row0 + ch * CB, CB)], sidx)
            pltpu.sync_copy(dst_hbm.at[pl.ds(row0 + ch * CB, CB)], didx)
            for b in range(2):
                pltpu.async_copy(x_hbm.at[sidx.at[b]], rows.at[b], sems[b])
            for i in range(CB):
                b = i % 2
                pltpu.make_async_copy(x_hbm.at[sidx.at[i]], rows.at[b],
                                      sems[b]).wait()
                pltpu.sync_copy(rows.at[b], acc.at[didx.at[i]], add=True)
                if i + 2 < CB:
                    pltpu.async_copy(x_hbm.at[sidx.at[i + 2]], rows.at[b],
                                     sems[b])
            return carry

        lax.fori_loop(0, NB // CB, body, 0)
        plsc.subcore_barrier()
        pltpu.sync_copy(acc.at[pl.ds(t * CHUNK, CHUNK)],
                        out_hbm.at[pl.ds(c * R + t * CHUNK, CHUNK)])

    return _sc_scatter


def _mm_bias_body(x_ref, w_ref, b_ref, o_ref):
    o_ref[...] = jnp.dot(x_ref[...], w_ref[...],
                         preferred_element_type=jnp.float32) + b_ref[...]


def _mm_bias(x, w, b, br):
    n = x.shape[0]
    return pl.pallas_call(
        _mm_bias_body,
        grid=(n // br,),
        in_specs=[pl.BlockSpec((br, x.shape[1]), lambda i: (i, 0)),
                  pl.BlockSpec((x.shape[1], HID), lambda i: (0, 0)),
                  pl.BlockSpec((1, HID), lambda i: (0, 0))],
        out_specs=pl.BlockSpec((br, HID), lambda i: (i, 0)),
        out_shape=jax.ShapeDtypeStruct((n, HID), jnp.float32),
    )(x, w, b)


def _mm_scale_body(x_ref, w_ref, deg_ref, o_ref):
    dinv = lax.rsqrt(deg_ref[...] + 1.0)
    o_ref[...] = dinv * jnp.dot(x_ref[...], w_ref[...],
                                preferred_element_type=jnp.float32)


def _mm_scale(x, w, degf):
    return pl.pallas_call(
        _mm_scale_body,
        grid=(2 * R // BR,),
        in_specs=[pl.BlockSpec((BR, HID), lambda i: (i, 0)),
                  pl.BlockSpec((HID, HID), lambda i: (0, 0)),
                  pl.BlockSpec((BR, HID), lambda i: (i, 0))],
        out_specs=pl.BlockSpec((BR, HID), lambda i: (i, 0)),
        out_shape=jax.ShapeDtypeStruct((2 * R, HID), jnp.float32),
    )(x, w, degf)


def _combine_body(s_ref, ht_ref, deg_ref, b_ref, w_ref, o_ref):
    dinv = lax.rsqrt(deg_ref[...] + 1.0)
    x = jnp.maximum(dinv * (s_ref[...] + ht_ref[...]) + b_ref[...], 0.0)
    o_ref[...] = dinv * jnp.dot(x, w_ref[...],
                                preferred_element_type=jnp.float32)


def _combine(s, ht, degf, b, w_next):
    return pl.pallas_call(
        _combine_body,
        grid=(2 * R // BR,),
        in_specs=[pl.BlockSpec((BR, HID), lambda i: (i, 0)),
                  pl.BlockSpec((BR, HID), lambda i: (i, 0)),
                  pl.BlockSpec((BR, HID), lambda i: (i, 0)),
                  pl.BlockSpec((1, HID), lambda i: (0, 0)),
                  pl.BlockSpec((HID, HID), lambda i: (0, 0))],
        out_specs=pl.BlockSpec((BR, HID), lambda i: (i, 0)),
        out_shape=jax.ShapeDtypeStruct((2 * R, HID), jnp.float32),
    )(s, ht, degf, b, w_next)


def _pool_body(m_ref, s_ref, ht_ref, deg_ref, b_ref, o_ref):
    i = pl.program_id(0)
    dinv = lax.rsqrt(deg_ref[...] + 1.0)
    y = dinv * (s_ref[...] + ht_ref[...]) + b_ref[...]

    @pl.when(i == 0)
    def _():
        o_ref[...] = jnp.zeros_like(o_ref)

    o_ref[...] += jnp.dot(m_ref[...], y, preferred_element_type=jnp.float32)


def _pool(s, ht, degf, b, mask):
    return pl.pallas_call(
        _pool_body,
        grid=(2 * R // BR,),
        in_specs=[pl.BlockSpec((8, BR), lambda i: (0, i)),
                  pl.BlockSpec((BR, HID), lambda i: (i, 0)),
                  pl.BlockSpec((BR, HID), lambda i: (i, 0)),
                  pl.BlockSpec((BR, HID), lambda i: (i, 0)),
                  pl.BlockSpec((1, HID), lambda i: (0, 0))],
        out_specs=pl.BlockSpec((8, HID), lambda i: (0, 0)),
        out_shape=jax.ShapeDtypeStruct((8, HID), jnp.float32),
    )(mask, s, ht, degf, b)


def kernel(mol_x, mol_edge_index, mol_batch, protein_x, protein_edge_index,
           batch_size, Wm, bm, Wp, bp, W1, b1, W2, b2, W3, b3):
    f32 = jnp.float32
    src_all, dst_all, mask, counts = _build_indices(
        mol_edge_index, mol_batch, protein_edge_index, batch_size)

    sc_scatter = _get_sc_scatter()
    zeros_chunk = jnp.zeros((CHUNK, HID), f32)
    ones_tab = jnp.ones((2 * R, HID), f32)

    degf = sc_scatter(ones_tab, src_all, dst_all, zeros_chunk)

    mol_xp = jnp.pad(mol_x.astype(f32), ((0, 0), (0, HID - mol_x.shape[1])))
    Wmp = jnp.pad(Wm.astype(f32), ((0, HID - Wm.shape[0]), (0, 0)))
    prot_xp = jnp.pad(protein_x.astype(f32),
                      ((0, 120), (0, HID - protein_x.shape[1])))
    Wpp = jnp.pad(Wp.astype(f32), ((0, HID - Wp.shape[0]), (0, 0)))
    mol_feats = _mm_bias(mol_xp, Wmp, bm.reshape(1, HID), 320)
    prot_feats = _mm_bias(prot_xp, Wpp, bp.reshape(1, HID), 640)[:NPROT]

    region = jnp.concatenate(
        [prot_feats, prot_feats, mol_feats, jnp.zeros((R - 10320, HID), f32)])
    x0 = jnp.concatenate([region, region])

    ht = _mm_scale(x0, W1.astype(f32), degf)
    s1 = sc_scatter(ht, src_all, dst_all, zeros_chunk)
    ht = _combine(s1, ht, degf, b1.reshape(1, HID), W2.astype(f32))
    s2 = sc_scatter(ht, src_all, dst_all, zeros_chunk)
    ht = _combine(s2, ht, degf, b2.reshape(1, HID), W3.astype(f32))
    s3 = sc_scatter(ht, src_all, dst_all, zeros_chunk)

    sums = _pool(s3, ht, degf, b3.reshape(1, HID), mask)[:G]
    return sums / counts[:, None]


# speedup vs baseline: 22.2382x; 1.0014x over previous
import functools

import jax
import jax.numpy as jnp
from jax import lax
from jax.experimental import pallas as pl
from jax.experimental.pallas import tpu as pltpu
from jax.experimental.pallas import tpu_sc as plsc

HID = 128
NPROT = 5000
G = 4
R = 10368
DUMMY = R - 1
NTILES = 16
CHUNK = R // NTILES
EBATCH = 128
NB = 80
CB = 16
PER_TILE = NB * EBATCH
E_PAD = NTILES * PER_TILE
BR = 6912


def _build_indices(mol_edge_index, mol_batch, protein_edge_index, batch_size):
    i32 = jnp.int32
    group = jnp.minimum(mol_batch, batch_size - 1).astype(i32)
    n_mol = group.shape[0]
    grp = jnp.arange(G, dtype=i32)
    cnt = jnp.sum((group[:, None] == grp[None, :]).astype(i32), axis=0)
    end = jnp.cumsum(cnt)
    start = end - cnt

    nodes = jnp.arange(n_mol, dtype=i32)
    mol_rows = (group // 2) * R + 10000 + nodes
    cent_local = (grp % 2) * NPROT + NPROT // 2
    cent_rows = (grp // 2) * R + cent_local

    ms, md = mol_edge_index[0].astype(i32), mol_edge_index[1].astype(i32)
    ie = group[ms]
    src_mm = (ie // 2) * R + 10000 + ms
    mclip = jnp.maximum(md, start[ie])
    is_mol = md < end[ie]
    dst_mm = jnp.where(is_mol, 10000 + mclip,
                       (ie % 2) * NPROT + (md - end[ie]))
    S_sm = jnp.concatenate([src_mm, mol_rows, cent_rows[group]])
    D_sm = jnp.concatenate([dst_mm, cent_local[group], 10000 + nodes])
    O_sm = jnp.concatenate([ie // 2, group // 2, group // 2])

    pe0 = protein_edge_index[0].astype(i32)
    pe1 = protein_edge_index[1].astype(i32)
    Ep = pe0.shape[0]
    n_sm = S_sm.shape[0]
    pad = E_PAD - (n_sm + 2 * Ep)
    NG = R - 10320
    spread_sm = 10320 + (jnp.arange(n_sm, dtype=i32) % NG)
    spread_pad = 10320 + (jnp.arange(pad, dtype=i32) % NG)

    src_sc, dst_sc = [], []
    for s in (0, 1):
        tiny_d = jnp.where(O_sm == s, D_sm, spread_sm)
        pad_src = s * R + 10320 + (jnp.arange(pad, dtype=i32) % NG)
        rest_s = jnp.concatenate(
            [(s * R) + pe0, (s * R + NPROT) + pe0, pad_src])
        rest_d = jnp.concatenate([pe1, NPROT + pe1, spread_pad])
        n_tiny = n_sm // NTILES
        n_rest = rest_s.shape[0] // NTILES
        src_sc.append(jnp.concatenate(
            [S_sm.reshape(NTILES, n_tiny), rest_s.reshape(NTILES, n_rest)],
            axis=1).reshape(-1))
        dst_sc.append(jnp.concatenate(
            [tiny_d.reshape(NTILES, n_tiny), rest_d.reshape(NTILES, n_rest)],
            axis=1).reshape(-1))
    src_all = jnp.concatenate(src_sc).reshape(2 * NTILES * NB, EBATCH)
    dst_all = jnp.concatenate(dst_sc).reshape(2 * NTILES * NB, EBATCH)

    mask = jnp.zeros((8, 2 * R), jnp.float32)
    for g in range(G):
        s = g // 2
        st = s * R + (g % 2) * NPROT
        mask = mask.at[g, st:st + NPROT].set(1.0)
        molm = (group == g).astype(jnp.float32)
        mask = lax.dynamic_update_slice(mask, molm[None, :],
                                        (g, s * R + 10000))
    counts = (cnt + NPROT).astype(jnp.float32)
    return src_all, dst_all, mask, counts


@functools.cache
def _get_sc_scatter():
    mesh = plsc.VectorSubcoreMesh(core_axis_name="c", subcore_axis_name="s")

    @functools.partial(
        pl.kernel,
        mesh=mesh,
        out_type=jax.ShapeDtypeStruct((2 * R, HID), jnp.float32),
        scratch_types=[
            pltpu.VMEM((CB, EBATCH), jnp.int32),
            pltpu.VMEM((CB, EBATCH), jnp.int32),
            pltpu.VMEM((2, EBATCH, HID), jnp.float32),
            pltpu.VMEM_SHARED((R, HID), jnp.float32),
            pltpu.SemaphoreType.DMA,
            pltpu.SemaphoreType.DMA,
        ],
    )
    def _sc_scatter(x_hbm, src_hbm, dst_hbm, zeros_hbm, out_hbm,
                    sidx, didx, rows, acc, sem0, sem1):
        c = lax.axis_index("c")
        t = lax.axis_index("s")
        row0 = (c * NTILES + t) * NB
        pltpu.sync_copy(zeros_hbm, acc.at[pl.ds(t * CHUNK, CHUNK)])
        plsc.subcore_barrier()

        sems = (sem0, sem1)

        def body(ch, carry):
            pltpu.sync_copy(src_hbm.at[pl.ds(row0 + ch * CB, CB)], sidx)
            pltpu.sync_copy(dst_hbm.at[pl.ds(row0 + ch * CB, CB)], didx)
            for b in range(2):
                pltpu.async_copy(x_hbm.at[sidx.at[b]], rows.at[b], sems[b])
            for i in range(CB):
                b = i % 2
                pltpu.make_async_copy(x_hbm.at[sidx.at[i]], rows.at[b],
                                      sems[b]).wait()
                pltpu.sync_copy(rows.at[b], acc.at[didx.at[i]], add=True)
                if i + 2 < CB:
                    pltpu.async_copy(x_hbm.at[sidx.at[i + 2]], rows.at[b],
                                     sems[b])
            return carry

        lax.fori_loop(0, NB // CB, body, 0)
        plsc.subcore_barrier()
        pltpu.sync_copy(acc.at[pl.ds(t * CHUNK, CHUNK)],
                        out_hbm.at[pl.ds(c * R + t * CHUNK, CHUNK)])

    return _sc_scatter


def _mm_bias_body(x_ref, w_ref, b_ref, o_ref):
    o_ref[...] = jnp.dot(x_ref[...], w_ref[...],
                         preferred_element_type=jnp.float32) + b_ref[...]


def _mm_bias(x, w, b, br):
    n = x.shape[0]
    return pl.pallas_call(
        _mm_bias_body,
        grid=(n // br,),
        in_specs=[pl.BlockSpec((br, x.shape[1]), lambda i: (i, 0)),
                  pl.BlockSpec((x.shape[1], HID), lambda i: (0, 0)),
                  pl.BlockSpec((1, HID), lambda i: (0, 0))],
        out_specs=pl.BlockSpec((br, HID), lambda i: (i, 0)),
        out_shape=jax.ShapeDtypeStruct((n, HID), jnp.float32),
    )(x, w, b)


def _mm_scale_body(x_ref, w_ref, deg_ref, o_ref):
    dinv = lax.rsqrt(deg_ref[...] + 1.0)
    o_ref[...] = dinv * jnp.dot(x_ref[...], w_ref[...],
                                preferred_element_type=jnp.float32)


def _mm_scale(x, w, degf):
    return pl.pallas_call(
        _mm_scale_body,
        grid=(2 * R // BR,),
        in_specs=[pl.BlockSpec((BR, HID), lambda i: (i, 0)),
                  pl.BlockSpec((HID, HID), lambda i: (0, 0)),
                  pl.BlockSpec((BR, HID), lambda i: (i, 0))],
        out_specs=pl.BlockSpec((BR, HID), lambda i: (i, 0)),
        out_shape=jax.ShapeDtypeStruct((2 * R, HID), jnp.float32),
    )(x, w, degf)


def _combine_body(s_ref, ht_ref, deg_ref, b_ref, w_ref, o_ref):
    dinv = lax.rsqrt(deg_ref[...] + 1.0)
    x = jnp.maximum(dinv * (s_ref[...] + ht_ref[...]) + b_ref[...], 0.0)
    o_ref[...] = dinv * jnp.dot(x, w_ref[...],
                                preferred_element_type=jnp.float32)


def _combine(s, ht, degf, b, w_next):
    return pl.pallas_call(
        _combine_body,
        grid=(2 * R // BR,),
        in_specs=[pl.BlockSpec((BR, HID), lambda i: (i, 0)),
                  pl.BlockSpec((BR, HID), lambda i: (i, 0)),
                  pl.BlockSpec((BR, HID), lambda i: (i, 0)),
                  pl.BlockSpec((1, HID), lambda i: (0, 0)),
                  pl.BlockSpec((HID, HID), lambda i: (0, 0))],
        out_specs=pl.BlockSpec((BR, HID), lambda i: (i, 0)),
        out_shape=jax.ShapeDtypeStruct((2 * R, HID), jnp.float32),
    )(s, ht, degf, b, w_next)


def _pool_body(m_ref, s_ref, ht_ref, deg_ref, b_ref, o_ref):
    i = pl.program_id(0)
    dinv = lax.rsqrt(deg_ref[...] + 1.0)
    y = dinv * (s_ref[...] + ht_ref[...]) + b_ref[...]

    @pl.when(i == 0)
    def _():
        o_ref[...] = jnp.zeros_like(o_ref)

    o_ref[...] += jnp.dot(m_ref[...], y, preferred_element_type=jnp.float32)


def _pool(s, ht, degf, b, mask):
    return pl.pallas_call(
        _pool_body,
        grid=(2 * R // BR,),
        in_specs=[pl.BlockSpec((8, BR), lambda i: (0, i)),
                  pl.BlockSpec((BR, HID), lambda i: (i, 0)),
                  pl.BlockSpec((BR, HID), lambda i: (i, 0)),
                  pl.BlockSpec((BR, HID), lambda i: (i, 0)),
                  pl.BlockSpec((1, HID), lambda i: (0, 0))],
        out_specs=pl.BlockSpec((8, HID), lambda i: (0, 0)),
        out_shape=jax.ShapeDtypeStruct((8, HID), jnp.float32),
    )(mask, s, ht, degf, b)


def kernel(mol_x, mol_edge_index, mol_batch, protein_x, protein_edge_index,
           batch_size, Wm, bm, Wp, bp, W1, b1, W2, b2, W3, b3):
    f32 = jnp.float32
    src_all, dst_all, mask, counts = _build_indices(
        mol_edge_index, mol_batch, protein_edge_index, batch_size)

    sc_scatter = _get_sc_scatter()
    zeros_chunk = jnp.zeros((CHUNK, HID), f32)
    ones_tab = jnp.ones((2 * R, HID), f32)

    degf = sc_scatter(ones_tab, src_all, dst_all, zeros_chunk)

    mol_xp = jnp.pad(mol_x.astype(f32), ((0, 0), (0, HID - mol_x.shape[1])))
    Wmp = jnp.pad(Wm.astype(f32), ((0, HID - Wm.shape[0]), (0, 0)))
    prot_xp = jnp.pad(protein_x.astype(f32),
                      ((0, 120), (0, HID - protein_x.shape[1])))
    Wpp = jnp.pad(Wp.astype(f32), ((0, HID - Wp.shape[0]), (0, 0)))
    mol_feats = _mm_bias(mol_xp, Wmp, bm.reshape(1, HID), 320)
    prot_feats = _mm_bias(prot_xp, Wpp, bp.reshape(1, HID), 640)[:NPROT]

    region = jnp.concatenate(
        [prot_feats, prot_feats, mol_feats, jnp.zeros((R - 10320, HID), f32)])
    x0 = jnp.concatenate([region, region])

    ht = _mm_scale(x0, W1.astype(f32), degf)
    s1 = sc_scatter(ht, src_all, dst_all, zeros_chunk)
    ht = _combine(s1, ht, degf, b1.reshape(1, HID), W2.astype(f32))
    s2 = sc_scatter(ht, src_all, dst_all, zeros_chunk)
    ht = _combine(s2, ht, degf, b2.reshape(1, HID), W3.astype(f32))
    s3 = sc_scatter(ht, src_all, dst_all, zeros_chunk)

    sums = _pool(s3, ht, degf, b3.reshape(1, HID), mask)[:G]
    return sums / counts[:, None]


# speedup vs baseline: 22.2392x; 1.0000x over previous
import functools

import jax
import jax.numpy as jnp
from jax import lax
from jax.experimental import pallas as pl
from jax.experimental.pallas import tpu as pltpu
from jax.experimental.pallas import tpu_sc as plsc

HID = 128
NPROT = 5000
G = 4
R = 10368
NTILES = 16
CHUNK = R // NTILES
EBATCH = 128
NB = 80
CB = 16
PER_TILE = NB * EBATCH
E_PAD = NTILES * PER_TILE
BR = 6912


def _build_indices(mol_edge_index, mol_batch, protein_edge_index, batch_size):
    i32 = jnp.int32
    group = jnp.minimum(mol_batch, batch_size - 1).astype(i32)
    n_mol = group.shape[0]
    grp = jnp.arange(G, dtype=i32)
    cnt = jnp.sum((group[:, None] == grp[None, :]).astype(i32), axis=0)
    end = jnp.cumsum(cnt)
    start = end - cnt

    nodes = jnp.arange(n_mol, dtype=i32)
    mol_rows = (group // 2) * R + 10000 + nodes
    cent_local = (grp % 2) * NPROT + NPROT // 2
    cent_rows = (grp // 2) * R + cent_local

    ms, md = mol_edge_index[0].astype(i32), mol_edge_index[1].astype(i32)
    ie = group[ms]
    src_mm = (ie // 2) * R + 10000 + ms
    mclip = jnp.maximum(md, start[ie])
    is_mol = md < end[ie]
    dst_mm = jnp.where(is_mol, 10000 + mclip,
                       (ie % 2) * NPROT + (md - end[ie]))
    S_sm = jnp.concatenate([src_mm, mol_rows, cent_rows[group]])
    D_sm = jnp.concatenate([dst_mm, cent_local[group], 10000 + nodes])
    O_sm = jnp.concatenate([ie // 2, group // 2, group // 2])

    pe0 = protein_edge_index[0].astype(i32)
    pe1 = protein_edge_index[1].astype(i32)
    Ep = pe0.shape[0]
    n_sm = S_sm.shape[0]
    pad = E_PAD - (n_sm + 2 * Ep)
    NG = R - 10320
    spread_sm = 10320 + (jnp.arange(n_sm, dtype=i32) % NG)
    spread_pad = 10320 + (jnp.arange(pad, dtype=i32) % NG)

    src_sc, dst_sc = [], []
    for s in (0, 1):
        tiny_d = jnp.where(O_sm == s, D_sm, spread_sm)
        pad_src = s * R + 10320 + (jnp.arange(pad, dtype=i32) % NG)
        rest_s = jnp.concatenate(
            [(s * R) + pe0, (s * R + NPROT) + pe0, pad_src])
        rest_d = jnp.concatenate([pe1, NPROT + pe1, spread_pad])
        n_tiny = n_sm // NTILES
        n_rest = rest_s.shape[0] // NTILES
        src_sc.append(jnp.concatenate(
            [S_sm.reshape(NTILES, n_tiny), rest_s.reshape(NTILES, n_rest)],
            axis=1).reshape(-1))
        dst_sc.append(jnp.concatenate(
            [tiny_d.reshape(NTILES, n_tiny), rest_d.reshape(NTILES, n_rest)],
            axis=1).reshape(-1))
    src_all = jnp.concatenate(src_sc).reshape(2 * NTILES * NB, EBATCH)
    dst_all = jnp.concatenate(dst_sc).reshape(2 * NTILES * NB, EBATCH)

    mask = jnp.zeros((8, 2 * R), jnp.float32)
    for g in range(G):
        s = g // 2
        st = s * R + (g % 2) * NPROT
        mask = mask.at[g, st:st + NPROT].set(1.0)
        molm = (group == g).astype(jnp.float32)
        mask = lax.dynamic_update_slice(mask, molm[None, :],
                                        (g, s * R + 10000))
    counts = (cnt + NPROT).astype(jnp.float32)
    return src_all, dst_all, mask, counts


@functools.cache
def _get_sc_scatter():
    mesh = plsc.VectorSubcoreMesh(core_axis_name="c", subcore_axis_name="s")

    @functools.partial(
        pl.kernel,
        mesh=mesh,
        out_type=jax.ShapeDtypeStruct((2 * R, HID), jnp.float32),
        scratch_types=[
            pltpu.VMEM((CB, EBATCH), jnp.int32),
            pltpu.VMEM((CB, EBATCH), jnp.int32),
            pltpu.VMEM((2, EBATCH, HID), jnp.float32),
            pltpu.VMEM_SHARED((R, HID), jnp.float32),
            pltpu.SemaphoreType.DMA,
            pltpu.SemaphoreType.DMA,
        ],
    )
    def _sc_scatter(x_hbm, src_hbm, dst_hbm, zeros_hbm, out_hbm,
                    sidx, didx, rows, acc, sem0, sem1):
        c = lax.axis_index("c")
        t = lax.axis_index("s")
        row0 = (c * NTILES + t) * NB
        pltpu.sync_copy(zeros_hbm, acc.at[pl.ds(t * CHUNK, CHUNK)])
        plsc.subcore_barrier()

        sems = (sem0, sem1)

        def body(ch, carry):
            pltpu.sync_copy(src_hbm.at[pl.ds(row0 + ch * CB, CB)], sidx)
            pltpu.sync_copy(dst_hbm.at[pl.ds(row0 + ch * CB, CB)], didx)
            for b in range(2):
                pltpu.async_copy(x_hbm.at[sidx.at[b]], rows.at[b], sems[b])
            for i in range(CB):
                b = i % 2
                pltpu.make_async_copy(x_hbm.at[sidx.at[i]], rows.at[b],
                                      sems[b]).wait()
                pltpu.sync_copy(rows.at[b], acc.at[didx.at[i]], add=True)
                if i + 2 < CB:
                    pltpu.async_copy(x_hbm.at[sidx.at[i + 2]], rows.at[b],
                                     sems[b])
            return carry

        lax.fori_loop(0, NB // CB, body, 0)
        plsc.subcore_barrier()
        pltpu.sync_copy(acc.at[pl.ds(t * CHUNK, CHUNK)],
                        out_hbm.at[pl.ds(c * R + t * CHUNK, CHUNK)])

    return _sc_scatter


def _mm_bias_body(x_ref, w_ref, b_ref, o_ref):
    o_ref[...] = jnp.dot(x_ref[...], w_ref[...],
                         preferred_element_type=jnp.float32) + b_ref[...]


def _mm_bias(x, w, b, br):
    n = x.shape[0]
    return pl.pallas_call(
        _mm_bias_body,
        grid=(n // br,),
        in_specs=[pl.BlockSpec((br, x.shape[1]), lambda i: (i, 0)),
                  pl.BlockSpec((x.shape[1], HID), lambda i: (0, 0)),
                  pl.BlockSpec((1, HID), lambda i: (0, 0))],
        out_specs=pl.BlockSpec((br, HID), lambda i: (i, 0)),
        out_shape=jax.ShapeDtypeStruct((n, HID), jnp.float32),
    )(x, w, b)


def _mm_scale_body(x_ref, w_ref, deg_ref, o_ref):
    dinv = lax.rsqrt(deg_ref[...] + 1.0)
    o_ref[...] = dinv * jnp.dot(x_ref[...], w_ref[...],
                                preferred_element_type=jnp.float32)


def _mm_scale(x, w, degf):
    return pl.pallas_call(
        _mm_scale_body,
        grid=(2 * R // BR,),
        in_specs=[pl.BlockSpec((BR, HID), lambda i: (i, 0)),
                  pl.BlockSpec((HID, HID), lambda i: (0, 0)),
                  pl.BlockSpec((BR, HID), lambda i: (i, 0))],
        out_specs=pl.BlockSpec((BR, HID), lambda i: (i, 0)),
        out_shape=jax.ShapeDtypeStruct((2 * R, HID), jnp.float32),
    )(x, w, degf)


def _combine_body(s_ref, ht_ref, deg_ref, b_ref, w_ref, o_ref):
    dinv = lax.rsqrt(deg_ref[...] + 1.0)
    x = jnp.maximum(dinv * (s_ref[...] + ht_ref[...]) + b_ref[...], 0.0)
    o_ref[...] = dinv * jnp.dot(x, w_ref[...],
                                preferred_element_type=jnp.float32)


def _combine(s, ht, degf, b, w_next):
    return pl.pallas_call(
        _combine_body,
        grid=(2 * R // BR,),
        in_specs=[pl.BlockSpec((BR, HID), lambda i: (i, 0)),
                  pl.BlockSpec((BR, HID), lambda i: (i, 0)),
                  pl.BlockSpec((BR, HID), lambda i: (i, 0)),
                  pl.BlockSpec((1, HID), lambda i: (0, 0)),
                  pl.BlockSpec((HID, HID), lambda i: (0, 0))],
        out_specs=pl.BlockSpec((BR, HID), lambda i: (i, 0)),
        out_shape=jax.ShapeDtypeStruct((2 * R, HID), jnp.float32),
    )(s, ht, degf, b, w_next)


def _pool_body(m_ref, s_ref, ht_ref, deg_ref, b_ref, o_ref):
    i = pl.program_id(0)
    dinv = lax.rsqrt(deg_ref[...] + 1.0)
    y = dinv * (s_ref[...] + ht_ref[...]) + b_ref[...]

    @pl.when(i == 0)
    def _():
        o_ref[...] = jnp.zeros_like(o_ref)

    o_ref[...] += jnp.dot(m_ref[...], y, preferred_element_type=jnp.float32)


def _pool(s, ht, degf, b, mask):
    return pl.pallas_call(
        _pool_body,
        grid=(2 * R // BR,),
        in_specs=[pl.BlockSpec((8, BR), lambda i: (0, i)),
                  pl.BlockSpec((BR, HID), lambda i: (i, 0)),
                  pl.BlockSpec((BR, HID), lambda i: (i, 0)),
                  pl.BlockSpec((BR, HID), lambda i: (i, 0)),
                  pl.BlockSpec((1, HID), lambda i: (0, 0))],
        out_specs=pl.BlockSpec((8, HID), lambda i: (0, 0)),
        out_shape=jax.ShapeDtypeStruct((8, HID), jnp.float32),
    )(mask, s, ht, degf, b)


def kernel(mol_x, mol_edge_index, mol_batch, protein_x, protein_edge_index,
           batch_size, Wm, bm, Wp, bp, W1, b1, W2, b2, W3, b3):
    f32 = jnp.float32
    src_all, dst_all, mask, counts = _build_indices(
        mol_edge_index, mol_batch, protein_edge_index, batch_size)

    sc_scatter = _get_sc_scatter()
    zeros_chunk = jnp.zeros((CHUNK, HID), f32)
    ones_tab = jnp.ones((2 * R, HID), f32)

    degf = sc_scatter(ones_tab, src_all, dst_all, zeros_chunk)

    mol_xp = jnp.pad(mol_x.astype(f32), ((0, 0), (0, HID - mol_x.shape[1])))
    Wmp = jnp.pad(Wm.astype(f32), ((0, HID - Wm.shape[0]), (0, 0)))
    prot_xp = jnp.pad(protein_x.astype(f32),
                      ((0, 120), (0, HID - protein_x.shape[1])))
    Wpp = jnp.pad(Wp.astype(f32), ((0, HID - Wp.shape[0]), (0, 0)))
    mol_feats = _mm_bias(mol_xp, Wmp, bm.reshape(1, HID), 320)
    prot_feats = _mm_bias(prot_xp, Wpp, bp.reshape(1, HID), 640)[:NPROT]

    region = jnp.concatenate(
        [prot_feats, prot_feats, mol_feats, jnp.zeros((R - 10320, HID), f32)])
    x0 = jnp.concatenate([region, region])

    ht = _mm_scale(x0, W1.astype(f32), degf)
    s1 = sc_scatter(ht, src_all, dst_all, zeros_chunk)
    ht = _combine(s1, ht, degf, b1.reshape(1, HID), W2.astype(f32))
    s2 = sc_scatter(ht, src_all, dst_all, zeros_chunk)
    ht = _combine(s2, ht, degf, b2.reshape(1, HID), W3.astype(f32))
    s3 = sc_scatter(ht, src_all, dst_all, zeros_chunk)

    sums = _pool(s3, ht, degf, b3.reshape(1, HID), mask)[:G]
    return sums / counts[:, None]
